# baseline (speedup 1.0000x reference)
import functools

import numpy as np
import jax
import jax.numpy as jnp
from jax import lax
from jax.experimental import pallas as pl
from jax.experimental.pallas import tpu as pltpu
from jax.experimental.pallas import tpu_sc as plsc

_RATIOS = np.array([0.5, 1.0, 2.0], dtype=np.float64)
_SCALES = np.array([1.0, 2.0 ** (1.0 / 3.0), 2.0 ** (2.0 / 3.0)], dtype=np.float64)
_HW = (64, 32, 16, 8)
_NA = 9

_base = 32.0 * np.tile(_SCALES, (2, len(_RATIOS))).T
_areas = _base[:, 0] * _base[:, 1]
_bw = np.sqrt(_areas / np.repeat(_RATIOS, len(_SCALES)))
_bh = _bw * np.repeat(_RATIOS, len(_SCALES))
_W_TAB = tuple(float(np.float32(v)) for v in _bw)
_H_TAB = tuple(float(np.float32(v)) for v in _bh)

_ROWS = tuple(h * h * _NA for h in _HW)
_NROWS = sum(_ROWS)
_NWORDS = _NROWS * 4
_NVREG = _NWORDS // 16
_VB = (0, 9216, 11520, 12096, 12240)
_ROW_OFF = (0, 36864, 46080, 48384)

_NC, _NS, _L = 2, 16, 16
_NW = _NC * _NS
_NV = 384
_STRIDE_NUM = _NVREG - _NV
_STEPS = tuple(float(np.float32(np.float64(w) / (w - 1))) for w in _HW)
_INV9 = float(np.float32(1.0 / 9.0))
_HALF_INV9 = float(np.float32(0.5 / 9.0))


def _gather16(tab, idx):
    dnums = lax.GatherDimensionNumbers(
        offset_dims=(), collapsed_slice_dims=(0,), start_index_map=(0,))
    return lax.gather(
        tab, idx[:, None], dnums, (1,),
        mode=lax.GatherScatterMode.PROMISE_IN_BOUNDS)


def _grid_body(out_hbm, buf):
    wid = lax.axis_index("s") * _NC + lax.axis_index("c")
    base = (wid * _STRIDE_NUM) // (_NW - 1)

    lane = lax.iota(jnp.int32, _L)
    dr = lax.shift_right_logical(lane, 2)
    c = lax.bitwise_and(lane, 3)
    is_y = c == 1
    is_w = c == 2
    c_le1 = c <= 1

    wtab = jnp.full((_L,), _W_TAB[8], jnp.float32)
    htab = jnp.full((_L,), _H_TAB[8], jnp.float32)
    for k in range(7, -1, -1):
        sel = lane <= k
        wtab = jnp.where(sel, _W_TAB[k], wtab)
        htab = jnp.where(sel, _H_TAB[k], htab)

    for lvl in range(4):
        scl = float(2.0 ** lvl)
        step = _STEPS[lvl]
        mask = _HW[lvl] - 1
        shift = (6, 5, 4, 3)[lvl]
        roff = _ROW_OFF[lvl]
        wtab_l = wtab * scl
        htab_l = htab * scl

        lo = jnp.maximum(base, _VB[lvl])
        hi = jnp.minimum(base + _NV, _VB[lvl + 1])

        def body(g, _, wtab_l=wtab_l, htab_l=htab_l, step=step,
                 mask=mask, shift=shift, roff=roff):
            rr = (g * 4 - roff) + dr
            rrf = rr.astype(jnp.float32)
            hwf = rrf * _INV9 + _HALF_INV9
            hw = hwf.astype(jnp.int32)
            a = rr - hw * _NA
            wi = lax.bitwise_and(hw, mask)
            hi_ = lax.shift_right_logical(hw, shift)
            cell = jnp.where(is_y, hi_, wi)
            xy = 0.5 + cell.astype(jnp.float32) * step
            aw = _gather16(wtab_l, a)
            ah = _gather16(htab_l, a)
            val = jnp.where(c_le1, xy, jnp.where(is_w, aw, ah))
            buf[pl.ds((g - base) * _L, _L)] = val
            return _

        lax.fori_loop(lo, hi, body, None)

    base_word = pl.multiple_of(base * _L, 16)
    pltpu.sync_copy(buf, out_hbm.at[pl.ds(base_word, _NV * _L)])


_sc_fn_cache = []


def _anchor_grid_sc():
    if not _sc_fn_cache:
        fn = functools.partial(
            pl.kernel,
            mesh=plsc.VectorSubcoreMesh(core_axis_name="c", subcore_axis_name="s"),
            out_type=jax.ShapeDtypeStruct((_NWORDS,), jnp.float32),
            scratch_types=[pltpu.VMEM((_NV * _L,), jnp.float32)],
        )(_grid_body)
        _sc_fn_cache.append(fn)
    return _sc_fn_cache[0]()


def kernel(feat0, feat1, feat2, feat3):
    del feat0, feat1, feat2, feat3
    flat = _anchor_grid_sc()
    return flat.reshape(_NROWS, 4)
